# baseline (speedup 1.0000x reference)
import functools

import jax
import jax.numpy as jnp
import numpy as np
from jax import lax
from jax.experimental import pallas as pl
from jax.experimental.pallas import tpu as pltpu
from jax.experimental.pallas import tpu_sc as plsc

GX, GY = 432, 496
SENT = GX * GY
NCELLS = SENT + 1
MAXV, MAXP = 16000, 32
NPTS = 200000
NPAD = 200704
KROWS, KCOLS = 1568, 128

W = 16
PPW = NPAD // W
NVREG = PPW // 16
CPW = 13408
NCV = CPW // 16
DEAD = 214527
BIGD = 1 << 28

VROWS = 512064
DUMPV = 512032
ZCH = 2048
CROWS = 16016
DUMPC = 16008
FLUSH = 112


def _tc_keys_body(xs_ref, ys_ref, zs_ref, keys_ref):
    x = xs_ref[...]
    y = ys_ref[...]
    z = zs_ref[...]
    xi = jnp.floor((x - jnp.float32(0.0)) / jnp.float32(0.16)).astype(jnp.int32)
    yi = jnp.floor((y - jnp.float32(-39.68)) / jnp.float32(0.16)).astype(jnp.int32)
    zi = jnp.floor((z - jnp.float32(-3.0)) / jnp.float32(4.0)).astype(jnp.int32)
    inr = ((xi >= 0) & (xi < GX) & (yi >= 0) & (yi < GY)
           & (zi >= 0) & (zi < 1))
    key = jnp.where(inr, xi * GY + yi, SENT)
    r = lax.broadcasted_iota(jnp.int32, (KROWS, KCOLS), 0)
    c = lax.broadcasted_iota(jnp.int32, (KROWS, KCOLS), 1)
    key = jnp.where(r * KCOLS + c >= NPTS, DEAD, key)
    keys_ref[...] = key


_tc_keys = pl.pallas_call(
    _tc_keys_body,
    out_shape=jax.ShapeDtypeStruct((KROWS, KCOLS), jnp.int32),
)


def _sc_body(pts8, keys, zsrc, cfill,
             vox8, crow,
             kbuf, cnt, dense, prow8, crowb, vidxb, gidxb, cidxb,
             tvec, tall, totals_sh, sems):
    w = lax.axis_index("s")
    lo_w = w * CPW
    hi_w = lo_w + CPW
    iota = lax.iota(jnp.int32, 16)
    z16 = jnp.zeros((16,), jnp.int32)
    dumpv16 = jnp.full((16,), DUMPV, jnp.int32)
    dumpc16 = jnp.full((16,), DUMPC, jnp.int32)

    SEM_G, SEM_S, SEM_C, SEM_K = 0, 1, 2, 3

    def reset_vslot(s):
        for j in range(8):
            plsc.store_scatter(vidxb, [z16 + s, j * 16 + iota], dumpv16)

    def reset_cslot(s):
        for j in range(8):
            plsc.store_scatter(cidxb, [z16 + s, j * 16 + iota], dumpc16)

    vbase = w * (VROWS // W)
    d0 = [pltpu.async_copy(zsrc, vox8.at[pl.ds(vbase + i * ZCH, ZCH)],
                           sems.at[0]) for i in range(15)]
    d0.append(pltpu.async_copy(zsrc.at[pl.ds(0, 1284)],
                               vox8.at[pl.ds(vbase + 15 * ZCH, 1284)],
                               sems.at[0]))
    cbase = w * (CROWS // W)
    d1 = [pltpu.async_copy(cfill, crow.at[pl.ds(cbase + i * 128, 128)],
                           sems.at[1]) for i in range(7)]
    d1.append(pltpu.async_copy(cfill.at[pl.ds(0, 105)],
                               crow.at[pl.ds(cbase + 7 * 128, 105)],
                               sems.at[1]))
    for s in range(3):
        reset_vslot(s)
        for j in range(8):
            plsc.store_scatter(gidxb, [z16 + s, j * 16 + iota], z16)
    for s in range(2):
        reset_cslot(s)
    for d in d0 + d1:
        d.wait()

    def zero_body(i, _):
        cnt[pl.ds(i * 16, 16)] = z16
        return 0
    lax.fori_loop(0, NCV, zero_body, 0)

    dk = pltpu.async_copy(keys.at[pl.ds(0, PPW)], kbuf.at[0], sems.at[SEM_K])
    for ch in range(W):
        dk.wait()
        if ch + 1 < W:
            dk = pltpu.async_copy(keys.at[pl.ds((ch + 1) * PPW, PPW)],
                                  kbuf.at[(ch + 1) % 2], sems.at[SEM_K])

        def cnt_body(i, _, ch=ch):
            k = kbuf[ch % 2, pl.ds(i * 16, 16)]
            m = (k >= lo_w) & (k < hi_w)

            @pl.when(jnp.any(m))
            def _():
                rel = jnp.where(m, k - lo_w, 0)
                c, lastm = plsc.scan_count(rel, m)
                base = plsc.load_gather(cnt, [rel], mask=m)
                plsc.store_scatter(cnt, [rel], base + c, mask=m & lastm)
            return 0
        lax.fori_loop(0, NVREG, cnt_body, 0)

    my_n = jnp.clip(NCELLS - lo_w, 0, CPW)

    def tot_body(i, acc):
        x = cnt[pl.ds(i * 16, 16)]
        occ = (x > 0) & (i * 16 + iota < my_n)
        return acc + jnp.sum(occ.astype(jnp.int32))
    total = lax.fori_loop(0, NCV, tot_body, jnp.int32(0))
    tvec[...] = jnp.full((16,), total, jnp.int32)
    pltpu.sync_copy(tvec.at[pl.ds(0, 8)], totals_sh.at[pl.ds(w * 8, 8)])
    plsc.subcore_barrier()
    pltpu.sync_copy(totals_sh, tall)
    tot = plsc.load_gather(tall, [iota * 8])
    base_w = jnp.sum(jnp.where(iota < w, tot, 0))

    def cflush_step(f):
        s = f % 2
        pltpu.async_copy(crowb.at[s], crow.at[cidxb.at[s]], sems.at[SEM_C])

        @pl.when(f >= 1)
        def _():
            pltpu.make_async_copy(crowb.at[0], crow.at[pl.ds(0, 128)],
                                  sems.at[SEM_C]).wait()
        reset_cslot((f + 1) % 2)

    def dense_body(i, carry):
        run, nacc, f = carry
        x = cnt[pl.ds(i * 16, 16)]
        cellv = lo_w + i * 16 + iota
        occ = (x > 0) & (cellv < NCELLS)
        oi = occ.astype(jnp.int32)
        cum = plsc.cumsum(oi)
        densev = run + cum - oi
        dense[pl.ds(i * 16, 16)] = jnp.where(occ, densev, BIGD)
        e = occ & (densev < MAXV)
        ei = e.astype(jnp.int32)
        pos = nacc + plsc.cumsum(ei) - 1
        s = f % 2
        gx = cellv // GY
        gy = cellv - gx * GY
        iss = cellv == SENT
        gyv = jnp.where(iss, GY, gy)
        gzv = iss.astype(jnp.int32)

        @pl.when(jnp.any(e))
        def _():
            plsc.store_scatter(crowb, [z16 + s, pos, z16], gx, mask=e)
            plsc.store_scatter(crowb, [z16 + s, pos, z16 + 1], gyv, mask=e)
            plsc.store_scatter(crowb, [z16 + s, pos, z16 + 2], gzv, mask=e)
            plsc.store_scatter(crowb, [z16 + s, pos, z16 + 3],
                               jnp.minimum(x, MAXP), mask=e)
            plsc.store_scatter(cidxb, [z16 + s, pos], densev, mask=e)
        nacc2 = nacc + jnp.sum(ei)
        do_flush = nacc2 >= FLUSH
        pl.when(do_flush)(lambda: cflush_step(f))
        return (run + jnp.sum(oi),
                jnp.where(do_flush, 0, nacc2),
                jnp.where(do_flush, f + 1, f))

    _, nacc, f = lax.fori_loop(0, NCV, dense_body,
                               (base_w, jnp.int32(0), jnp.int32(0)))
    pl.when(nacc > 0)(lambda: cflush_step(f))
    fc = jnp.where(nacc > 0, f + 1, f)

    @pl.when(fc >= 1)
    def _():
        pltpu.make_async_copy(crowb.at[0], crow.at[pl.ds(0, 128)],
                              sems.at[SEM_C]).wait()

    lax.fori_loop(0, NCV, zero_body, 0)

    def vflush_step(f):
        s = f % 3
        pltpu.async_copy(pts8.at[gidxb.at[s]], prow8.at[s], sems.at[SEM_G])

        @pl.when(f >= 1)
        def _():
            s1 = (f - 1) % 3
            pltpu.make_async_copy(pts8.at[pl.ds(0, 128)], prow8.at[0],
                                  sems.at[SEM_G]).wait()
            pltpu.async_copy(prow8.at[s1], vox8.at[vidxb.at[s1]],
                             sems.at[SEM_S])

        @pl.when(f >= 2)
        def _():
            pltpu.make_async_copy(prow8.at[0], vox8.at[pl.ds(0, 128)],
                                  sems.at[SEM_S]).wait()
        reset_vslot((f + 1) % 3)

    def make_pts_body(ch):
        def pts_body(i, carry):
            nacc, f = carry
            k = kbuf[ch % 2, pl.ds(i * 16, 16)]
            m = (k >= lo_w) & (k < hi_w)

            def heavy():
                rel = jnp.where(m, k - lo_w, 0)
                c, lastm = plsc.scan_count(rel, m)
                base = plsc.load_gather(cnt, [rel], mask=m)
                plsc.store_scatter(cnt, [rel], base + c, mask=m & lastm)
                rank = base + c - 1
                dv = plsc.load_gather(dense, [rel], mask=m)
                valid = m & (rank < MAXP) & (dv < MAXV)
                vi = valid.astype(jnp.int32)
                pos = nacc + plsc.cumsum(vi) - 1
                s = f % 3
                plsc.store_scatter(vidxb, [z16 + s, pos], dv * MAXP + rank,
                                   mask=valid)
                plsc.store_scatter(gidxb, [z16 + s, pos],
                                   ch * PPW + i * 16 + iota, mask=valid)
                return jnp.sum(vi)

            nadd = lax.cond(jnp.any(m), heavy, lambda: jnp.int32(0))
            nacc2 = nacc + nadd
            do_flush = nacc2 >= FLUSH
            pl.when(do_flush)(lambda: vflush_step(f))
            return (jnp.where(do_flush, 0, nacc2),
                    jnp.where(do_flush, f + 1, f))
        return pts_body

    carry = (jnp.int32(0), jnp.int32(0))
    dk = pltpu.async_copy(keys.at[pl.ds(0, PPW)], kbuf.at[0], sems.at[SEM_K])
    for ch in range(W):
        dk.wait()
        if ch + 1 < W:
            dk = pltpu.async_copy(keys.at[pl.ds((ch + 1) * PPW, PPW)],
                                  kbuf.at[(ch + 1) % 2], sems.at[SEM_K])
        carry = lax.fori_loop(0, NVREG, make_pts_body(ch), carry)

    nacc, f = carry
    pl.when(nacc > 0)(lambda: vflush_step(f))
    fv = jnp.where(nacc > 0, f + 1, f)

    @pl.when(fv >= 1)
    def _():
        s1 = (fv - 1) % 3
        pltpu.make_async_copy(pts8.at[pl.ds(0, 128)], prow8.at[0],
                              sems.at[SEM_G]).wait()
        pltpu.async_copy(prow8.at[s1], vox8.at[vidxb.at[s1]],
                         sems.at[SEM_S])
        pltpu.make_async_copy(prow8.at[0], vox8.at[pl.ds(0, 128)],
                              sems.at[SEM_S]).wait()

    @pl.when(fv >= 2)
    def _():
        pltpu.make_async_copy(prow8.at[0], vox8.at[pl.ds(0, 128)],
                              sems.at[SEM_S]).wait()


_sc_mesh = plsc.VectorSubcoreMesh(
    core_axis_name="c", subcore_axis_name="s", num_cores=1)

_sc_vox = pl.kernel(
    _sc_body,
    out_type=[jax.ShapeDtypeStruct((VROWS, 8), jnp.float32),
              jax.ShapeDtypeStruct((CROWS, 8), jnp.int32)],
    mesh=_sc_mesh,
    compiler_params=pltpu.CompilerParams(
        needs_layout_passes=False, use_tc_tiling_on_sc=False),
    scratch_types=[pltpu.VMEM((2, PPW), jnp.int32),
                   pltpu.VMEM((CPW,), jnp.int32),
                   pltpu.VMEM((CPW,), jnp.int32),
                   pltpu.VMEM((3, 128, 8), jnp.float32),
                   pltpu.VMEM((2, 128, 8), jnp.int32),
                   pltpu.VMEM((3, 128), jnp.int32),
                   pltpu.VMEM((3, 128), jnp.int32),
                   pltpu.VMEM((2, 128), jnp.int32),
                   pltpu.VMEM((16,), jnp.int32),
                   pltpu.VMEM((128,), jnp.int32),
                   pltpu.VMEM_SHARED((128,), jnp.int32),
                   pltpu.SemaphoreType.DMA((4,))],
)


def kernel(points):
    pts8 = jnp.pad(points, ((0, NPAD - NPTS), (0, 4)))
    soa = jnp.transpose(jnp.pad(points, ((0, NPAD - NPTS), (0, 0))))
    xs = soa[0].reshape(KROWS, KCOLS)
    ys = soa[1].reshape(KROWS, KCOLS)
    zs = soa[2].reshape(KROWS, KCOLS)
    keys = _tc_keys(xs, ys, zs).reshape(NPAD)

    zsrc = np.zeros((ZCH, 8), np.float32)
    cfill = np.broadcast_to(
        np.array([GX, GY, 1, 0, 0, 0, 0, 0], np.int32), (128, 8)).copy()

    vox8, crow = _sc_vox(pts8, keys, zsrc, cfill)
    voxels = vox8[:MAXV * MAXP, :4].reshape(MAXV, MAXP, 4)
    coordinates = crow[:MAXV, :3]
    num_points_per_voxel = crow[:MAXV, 3]
    return voxels, coordinates, num_points_per_voxel


# speedup vs baseline: 1.2074x; 1.2074x over previous
import functools

import jax
import jax.numpy as jnp
import numpy as np
from jax import lax
from jax.experimental import pallas as pl
from jax.experimental.pallas import tpu as pltpu
from jax.experimental.pallas import tpu_sc as plsc

GX, GY = 432, 496
SENT = GX * GY
NCELLS = SENT + 1
MAXV, MAXP = 16000, 32
NPTS = 200000
NPAD = 200704
KROWS, KCOLS = 1568, 128

W = 16
PPW = NPAD // W
NVREG = PPW // 16
CPW = 13408
NCV = CPW // 16
DEAD = 214527
BIGD = 1 << 28

VROWS = 512064
DUMPV = 512032
ZCH = 2048
CROWS = 16016
DUMPC = 16008
FLUSH = 112


def _tc_keys_body(xs_ref, ys_ref, zs_ref, keys_ref):
    x = xs_ref[...]
    y = ys_ref[...]
    z = zs_ref[...]
    xi = jnp.floor((x - jnp.float32(0.0)) / jnp.float32(0.16)).astype(jnp.int32)
    yi = jnp.floor((y - jnp.float32(-39.68)) / jnp.float32(0.16)).astype(jnp.int32)
    zi = jnp.floor((z - jnp.float32(-3.0)) / jnp.float32(4.0)).astype(jnp.int32)
    inr = ((xi >= 0) & (xi < GX) & (yi >= 0) & (yi < GY)
           & (zi >= 0) & (zi < 1))
    key = jnp.where(inr, xi * GY + yi, SENT)
    r = lax.broadcasted_iota(jnp.int32, (KROWS, KCOLS), 0)
    c = lax.broadcasted_iota(jnp.int32, (KROWS, KCOLS), 1)
    key = jnp.where(r * KCOLS + c >= NPTS, DEAD, key)
    keys_ref[...] = key


_tc_keys = pl.pallas_call(
    _tc_keys_body,
    out_shape=jax.ShapeDtypeStruct((KROWS, KCOLS), jnp.int32),
)


def _sc_body(pts8, keys, zsrc, cfill,
             vox8, crow,
             kbuf, cnt, dense, prow8, crowb, vidxb, gidxb, cidxb,
             tvec, tall, totals_sh, sems):
    w = lax.axis_index("s")
    lo_w = w * CPW
    hi_w = lo_w + CPW
    iota = lax.iota(jnp.int32, 16)
    z16 = jnp.zeros((16,), jnp.int32)
    dumpv16 = jnp.full((16,), DUMPV, jnp.int32)
    dumpc16 = jnp.full((16,), DUMPC, jnp.int32)

    SEM_G, SEM_S, SEM_C, SEM_K = 0, 1, 2, 3

    def reset_vslot(s):
        for j in range(8):
            plsc.store_scatter(vidxb, [z16 + s, j * 16 + iota], dumpv16)

    def reset_cslot(s):
        for j in range(8):
            plsc.store_scatter(cidxb, [z16 + s, j * 16 + iota], dumpc16)

    vbase = w * (VROWS // W)
    d0 = [pltpu.async_copy(zsrc, vox8.at[pl.ds(vbase + i * ZCH, ZCH)],
                           sems.at[0]) for i in range(15)]
    d0.append(pltpu.async_copy(zsrc.at[pl.ds(0, 1284)],
                               vox8.at[pl.ds(vbase + 15 * ZCH, 1284)],
                               sems.at[0]))
    cbase = w * (CROWS // W)
    d1 = [pltpu.async_copy(cfill, crow.at[pl.ds(cbase + i * 128, 128)],
                           sems.at[1]) for i in range(7)]
    d1.append(pltpu.async_copy(cfill.at[pl.ds(0, 105)],
                               crow.at[pl.ds(cbase + 7 * 128, 105)],
                               sems.at[1]))
    for s in range(3):
        reset_vslot(s)
        for j in range(8):
            plsc.store_scatter(gidxb, [z16 + s, j * 16 + iota], z16)
    for s in range(2):
        reset_cslot(s)
    for d in d0 + d1:
        d.wait()

    def zero_body(i, _):
        cnt[pl.ds(i * 16, 16)] = z16
        return 0
    lax.fori_loop(0, NCV, zero_body, 0)

    dk = pltpu.async_copy(keys.at[pl.ds(0, PPW)], kbuf.at[0], sems.at[SEM_K])
    for ch in range(W):
        dk.wait()
        if ch + 1 < W:
            dk = pltpu.async_copy(keys.at[pl.ds((ch + 1) * PPW, PPW)],
                                  kbuf.at[(ch + 1) % 2], sems.at[SEM_K])

        def cnt_body(i, _, ch=ch):
            k = kbuf[ch % 2, pl.ds(i * 16, 16)]
            m = (k >= lo_w) & (k < hi_w)
            rel = jnp.where(m, k - lo_w, 0)
            c, lastm = plsc.scan_count(rel, m)
            base = plsc.load_gather(cnt, [rel], mask=m)
            plsc.store_scatter(cnt, [rel], base + c, mask=m & lastm)
            return 0
        lax.fori_loop(0, NVREG, cnt_body, 0)

    my_n = jnp.clip(NCELLS - lo_w, 0, CPW)

    def tot_body(i, acc):
        x = cnt[pl.ds(i * 16, 16)]
        occ = (x > 0) & (i * 16 + iota < my_n)
        return acc + jnp.sum(occ.astype(jnp.int32))
    total = lax.fori_loop(0, NCV, tot_body, jnp.int32(0))
    tvec[...] = jnp.full((16,), total, jnp.int32)
    pltpu.sync_copy(tvec.at[pl.ds(0, 8)], totals_sh.at[pl.ds(w * 8, 8)])
    plsc.subcore_barrier()
    pltpu.sync_copy(totals_sh, tall)
    tot = plsc.load_gather(tall, [iota * 8])
    base_w = jnp.sum(jnp.where(iota < w, tot, 0))

    def cflush_step(f):
        s = f % 2
        pltpu.async_copy(crowb.at[s], crow.at[cidxb.at[s]], sems.at[SEM_C])

        @pl.when(f >= 1)
        def _():
            pltpu.make_async_copy(crowb.at[0], crow.at[pl.ds(0, 128)],
                                  sems.at[SEM_C]).wait()
        reset_cslot((f + 1) % 2)

    def dense_body(i, carry):
        run, nacc, f = carry
        x = cnt[pl.ds(i * 16, 16)]
        cellv = lo_w + i * 16 + iota
        occ = (x > 0) & (cellv < NCELLS)
        oi = occ.astype(jnp.int32)
        cum = plsc.cumsum(oi)
        densev = run + cum - oi
        dense[pl.ds(i * 16, 16)] = jnp.where(occ, densev, BIGD)
        e = occ & (densev < MAXV)
        ei = e.astype(jnp.int32)
        pos = nacc + plsc.cumsum(ei) - 1
        s = f % 2
        gx = cellv // GY
        gy = cellv - gx * GY
        iss = cellv == SENT
        gyv = jnp.where(iss, GY, gy)
        gzv = iss.astype(jnp.int32)

        plsc.store_scatter(crowb, [z16 + s, pos, z16], gx, mask=e)
        plsc.store_scatter(crowb, [z16 + s, pos, z16 + 1], gyv, mask=e)
        plsc.store_scatter(crowb, [z16 + s, pos, z16 + 2], gzv, mask=e)
        plsc.store_scatter(crowb, [z16 + s, pos, z16 + 3],
                           jnp.minimum(x, MAXP), mask=e)
        plsc.store_scatter(cidxb, [z16 + s, pos], densev, mask=e)
        nacc2 = nacc + jnp.sum(ei)
        do_flush = nacc2 >= FLUSH
        pl.when(do_flush)(lambda: cflush_step(f))
        return (run + jnp.sum(oi),
                jnp.where(do_flush, 0, nacc2),
                jnp.where(do_flush, f + 1, f))

    _, nacc, f = lax.fori_loop(0, NCV, dense_body,
                               (base_w, jnp.int32(0), jnp.int32(0)))
    pl.when(nacc > 0)(lambda: cflush_step(f))
    fc = jnp.where(nacc > 0, f + 1, f)

    @pl.when(fc >= 1)
    def _():
        pltpu.make_async_copy(crowb.at[0], crow.at[pl.ds(0, 128)],
                              sems.at[SEM_C]).wait()

    lax.fori_loop(0, NCV, zero_body, 0)

    def vflush_step(f):
        s = f % 3
        pltpu.async_copy(pts8.at[gidxb.at[s]], prow8.at[s], sems.at[SEM_G])

        @pl.when(f >= 1)
        def _():
            s1 = (f - 1) % 3
            pltpu.make_async_copy(pts8.at[pl.ds(0, 128)], prow8.at[0],
                                  sems.at[SEM_G]).wait()
            pltpu.async_copy(prow8.at[s1], vox8.at[vidxb.at[s1]],
                             sems.at[SEM_S])

        @pl.when(f >= 2)
        def _():
            pltpu.make_async_copy(prow8.at[0], vox8.at[pl.ds(0, 128)],
                                  sems.at[SEM_S]).wait()
        reset_vslot((f + 1) % 3)

    def make_pts_body(ch):
        def pts_body(i, carry):
            nacc, f = carry
            k = kbuf[ch % 2, pl.ds(i * 16, 16)]
            m = (k >= lo_w) & (k < hi_w)

            rel = jnp.where(m, k - lo_w, 0)
            c, lastm = plsc.scan_count(rel, m)
            base = plsc.load_gather(cnt, [rel], mask=m)
            plsc.store_scatter(cnt, [rel], base + c, mask=m & lastm)
            rank = base + c - 1
            dv = plsc.load_gather(dense, [rel], mask=m)
            valid = m & (rank < MAXP) & (dv < MAXV)
            vi = valid.astype(jnp.int32)
            pos = nacc + plsc.cumsum(vi) - 1
            s = f % 3
            plsc.store_scatter(vidxb, [z16 + s, pos], dv * MAXP + rank,
                               mask=valid)
            plsc.store_scatter(gidxb, [z16 + s, pos],
                               ch * PPW + i * 16 + iota, mask=valid)
            nacc2 = nacc + jnp.sum(vi)
            do_flush = nacc2 >= FLUSH
            pl.when(do_flush)(lambda: vflush_step(f))
            return (jnp.where(do_flush, 0, nacc2),
                    jnp.where(do_flush, f + 1, f))
        return pts_body

    carry = (jnp.int32(0), jnp.int32(0))
    dk = pltpu.async_copy(keys.at[pl.ds(0, PPW)], kbuf.at[0], sems.at[SEM_K])
    for ch in range(W):
        dk.wait()
        if ch + 1 < W:
            dk = pltpu.async_copy(keys.at[pl.ds((ch + 1) * PPW, PPW)],
                                  kbuf.at[(ch + 1) % 2], sems.at[SEM_K])
        carry = lax.fori_loop(0, NVREG, make_pts_body(ch), carry)

    nacc, f = carry
    pl.when(nacc > 0)(lambda: vflush_step(f))
    fv = jnp.where(nacc > 0, f + 1, f)

    @pl.when(fv >= 1)
    def _():
        s1 = (fv - 1) % 3
        pltpu.make_async_copy(pts8.at[pl.ds(0, 128)], prow8.at[0],
                              sems.at[SEM_G]).wait()
        pltpu.async_copy(prow8.at[s1], vox8.at[vidxb.at[s1]],
                         sems.at[SEM_S])
        pltpu.make_async_copy(prow8.at[0], vox8.at[pl.ds(0, 128)],
                              sems.at[SEM_S]).wait()

    @pl.when(fv >= 2)
    def _():
        pltpu.make_async_copy(prow8.at[0], vox8.at[pl.ds(0, 128)],
                              sems.at[SEM_S]).wait()


_sc_mesh = plsc.VectorSubcoreMesh(
    core_axis_name="c", subcore_axis_name="s", num_cores=1)

_sc_vox = pl.kernel(
    _sc_body,
    out_type=[jax.ShapeDtypeStruct((VROWS, 8), jnp.float32),
              jax.ShapeDtypeStruct((CROWS, 8), jnp.int32)],
    mesh=_sc_mesh,
    compiler_params=pltpu.CompilerParams(
        needs_layout_passes=False, use_tc_tiling_on_sc=False),
    scratch_types=[pltpu.VMEM((2, PPW), jnp.int32),
                   pltpu.VMEM((CPW,), jnp.int32),
                   pltpu.VMEM((CPW,), jnp.int32),
                   pltpu.VMEM((3, 128, 8), jnp.float32),
                   pltpu.VMEM((2, 128, 8), jnp.int32),
                   pltpu.VMEM((3, 128), jnp.int32),
                   pltpu.VMEM((3, 128), jnp.int32),
                   pltpu.VMEM((2, 128), jnp.int32),
                   pltpu.VMEM((16,), jnp.int32),
                   pltpu.VMEM((128,), jnp.int32),
                   pltpu.VMEM_SHARED((128,), jnp.int32),
                   pltpu.SemaphoreType.DMA((4,))],
)


def kernel(points):
    pts8 = jnp.pad(points, ((0, NPAD - NPTS), (0, 4)))
    soa = jnp.transpose(jnp.pad(points, ((0, NPAD - NPTS), (0, 0))))
    xs = soa[0].reshape(KROWS, KCOLS)
    ys = soa[1].reshape(KROWS, KCOLS)
    zs = soa[2].reshape(KROWS, KCOLS)
    keys = _tc_keys(xs, ys, zs).reshape(NPAD)

    zsrc = np.zeros((ZCH, 8), np.float32)
    cfill = np.broadcast_to(
        np.array([GX, GY, 1, 0, 0, 0, 0, 0], np.int32), (128, 8)).copy()

    vox8, crow = _sc_vox(pts8, keys, zsrc, cfill)
    voxels = vox8[:MAXV * MAXP, :4].reshape(MAXV, MAXP, 4)
    coordinates = crow[:MAXV, :3]
    num_points_per_voxel = crow[:MAXV, 3]
    return voxels, coordinates, num_points_per_voxel
